# Initial kernel scaffold; baseline (speedup 1.0000x reference)
import jax
import jax.numpy as jnp
from jax.experimental import pallas as pl


def kernel(users, pos_items, neg_items, mask, user_emb, item_emb, adj_row, adj_col, adj_val):
    raise NotImplementedError("write your pallas kernel here")


# speedup vs baseline: 3.9465x; 3.9465x over previous
import functools

import jax
import jax.numpy as jnp
from jax import lax
from jax.experimental import pallas as pl
from jax.experimental.pallas import tpu as pltpu
from jax.experimental.pallas import tpu_sc as plsc

N_USER = 25000
N_ITEM = 25000
N = N_USER + N_ITEM
EMB = 64
NNZ = 800000

NNZ_PAD = 819200
W = 32
EPW = NNZ_PAD // W
BLK = 512
NBLK = EPW // BLK
NBATCH = BLK // 128
SLOTS = 6144
SROWS = 6400
MAP_PAD = 50048

_mesh = plsc.VectorSubcoreMesh(core_axis_name="c", subcore_axis_name="s")


@functools.partial(
    pl.kernel,
    out_type=(
        jax.ShapeDtypeStruct((2, SROWS, EMB), jnp.float32),
        jax.ShapeDtypeStruct((SLOTS,), jnp.int32),
    ),
    mesh=_mesh,
    compiler_params=pltpu.CompilerParams(needs_layout_passes=False, use_tc_tiling_on_sc=False),
    scratch_types=[
        pltpu.VMEM((MAP_PAD,), jnp.int32),
        pltpu.VMEM((SLOTS,), jnp.int32),
        pltpu.VMEM((NBATCH, 128), jnp.int32),
        pltpu.VMEM((NBATCH, 128), jnp.int32),
        pltpu.VMEM((NBATCH, 128), jnp.int32),
        pltpu.VMEM((NBATCH, 128), jnp.float32),
        pltpu.VMEM((BLK, EMB), jnp.float32),
        pltpu.VMEM_SHARED((SROWS, EMB), jnp.float32),
        pltpu.SemaphoreType.DMA,
    ],
)
def _spmv_kernel(ego_hbm, slots_hbm, row_hbm, col_hbm, val_hbm,
                 spart_hbm, cid_hbm,
                 map_v, slots_v, row_v, col_v, t_v, val_v, rows_v,
                 acc_sh, sem):
    cidx = lax.axis_index("c")
    sid = lax.axis_index("s")
    wid = sid * 2 + cidx

    pltpu.sync_copy(slots_hbm, slots_v)

    def _init_map(i, carry):
        map_v[pl.ds(i * 16, 16)] = jnp.full((16,), -1, jnp.int32)
        return carry
    lax.fori_loop(0, MAP_PAD // 16, _init_map, 0)

    lane = jnp.arange(16, dtype=jnp.int32)

    def _scat_map(j, carry):
        nodes = slots_v[pl.ds(j * 16, 16)]
        plsc.store_scatter(map_v, [nodes], lane + j * 16)
        return carry
    lax.fori_loop(0, SLOTS // 16, _scat_map, 0)

    zrows = SROWS // 16

    def _zero_row(i, carry):
        for c4 in range(EMB // 16):
            rows_v[i, pl.ds(c4 * 16, 16)] = jnp.zeros((16,), jnp.float32)
        return carry
    lax.fori_loop(0, zrows, _zero_row, 0)
    pltpu.sync_copy(rows_v.at[pl.ds(0, zrows)],
                    acc_sh.at[pl.ds(sid * zrows, zrows)])
    plsc.subcore_barrier()

    rbase = wid * (EPW // 128)

    def _block(b, carry):
        roff = rbase + b * NBATCH
        pltpu.sync_copy(row_hbm.at[pl.ds(roff, NBATCH)], row_v)
        pltpu.sync_copy(col_hbm.at[pl.ds(roff, NBATCH)], col_v)
        pltpu.sync_copy(val_hbm.at[pl.ds(roff, NBATCH)], val_v)

        def _tgrp(i, c2):
            j = i // 8
            k = i % 8
            r16 = row_v[j, pl.ds(k * 16, 16)]
            t16 = plsc.load_gather(map_v, [r16])
            t_v[j, pl.ds(k * 16, 16)] = jnp.where(t16 >= 0, t16, SLOTS)
            return c2
        lax.fori_loop(0, BLK // 16, _tgrp, 0)

        cps = [pltpu.async_copy(ego_hbm.at[col_v.at[j]],
                                rows_v.at[pl.ds(j * 128, 128)], sem)
               for j in range(NBATCH)]
        for cp in cps:
            cp.wait()

        def _scale(i, c2):
            j = i // 8
            k = i % 8
            v16 = val_v[j, pl.ds(k * 16, 16)]
            for m in range(16):
                r = i * 16 + m
                for c4 in range(EMB // 16):
                    sl = pl.ds(c4 * 16, 16)
                    rows_v[r, sl] = rows_v[r, sl] * v16[m]
            return c2
        lax.fori_loop(0, BLK // 16, _scale, 0)

        for j in range(NBATCH):
            pltpu.sync_copy(rows_v.at[pl.ds(j * 128, 128)],
                            acc_sh.at[t_v.at[j]], add=True)
        return carry
    lax.fori_loop(0, NBLK, _block, 0)

    plsc.subcore_barrier()
    pltpu.sync_copy(acc_sh.at[pl.ds(sid * zrows, zrows)],
                    rows_v.at[pl.ds(0, zrows)])
    pltpu.sync_copy(rows_v.at[pl.ds(0, zrows)],
                    spart_hbm.at[cidx].at[pl.ds(sid * zrows, zrows)])

    @pl.when(wid == 0)
    def _():
        def _cgrp(j, carry):
            nodes = slots_v[pl.ds(j * 16, 16)]
            slots_v[pl.ds(j * 16, 16)] = plsc.load_gather(map_v, [nodes])
            return carry
        lax.fori_loop(0, SLOTS // 16, _cgrp, 0)
        pltpu.sync_copy(slots_v, cid_hbm)


SPW = SLOTS // W


@functools.partial(
    pl.kernel,
    out_type=jax.ShapeDtypeStruct((SLOTS, EMB), jnp.float32),
    mesh=_mesh,
    compiler_params=pltpu.CompilerParams(needs_layout_passes=False, use_tc_tiling_on_sc=False),
    scratch_types=[
        pltpu.VMEM((2, 96), jnp.int32),
        pltpu.VMEM((2, 96), jnp.int32),
        pltpu.VMEM((SPW, EMB), jnp.float32),
        pltpu.VMEM((SPW, EMB), jnp.float32),
        pltpu.VMEM((SPW, EMB), jnp.float32),
        pltpu.SemaphoreType.DMA,
    ],
)
def _assemble_kernel(ego_hbm, slots_hbm, cid_hbm, spart_hbm,
                     out_hbm, idx_v, cid_v, ego_r, s0_r, s1_r, sem):
    cidx = lax.axis_index("c")
    sid = lax.axis_index("s")
    wid = sid * 2 + cidx
    base = wid * 2

    pltpu.sync_copy(slots_hbm.at[pl.ds(base, 2)], idx_v)
    pltpu.sync_copy(cid_hbm.at[pl.ds(base, 2)], cid_v)

    cps = []
    for h in range(2):
        dst = pl.ds(h * 96, 96)
        cps.append(pltpu.async_copy(ego_hbm.at[idx_v.at[h]],
                                    ego_r.at[dst], sem))
        cps.append(pltpu.async_copy(spart_hbm.at[0].at[cid_v.at[h]],
                                    s0_r.at[dst], sem))
        cps.append(pltpu.async_copy(spart_hbm.at[1].at[cid_v.at[h]],
                                    s1_r.at[dst], sem))
    for cp in cps:
        cp.wait()

    def _comb(i, carry):
        for c4 in range(EMB // 16):
            sl = pl.ds(c4 * 16, 16)
            ego_r[i, sl] = (ego_r[i, sl]
                            + 3.0 * (s0_r[i, sl] + s1_r[i, sl])) * 0.25
        return carry
    lax.fori_loop(0, SPW, _comb, 0)

    pltpu.sync_copy(ego_r, out_hbm.at[pl.ds(wid * SPW, SPW)])


def kernel(users, pos_items, neg_items, mask, user_emb, item_emb,
           adj_row, adj_col, adj_val):
    del mask
    ego = jnp.concatenate([user_emb, item_emb], axis=0)
    slots = jnp.concatenate(
        [users, pos_items + N_USER, neg_items + N_USER]).astype(jnp.int32)

    pad = NNZ_PAD - NNZ
    rowp = jnp.concatenate(
        [adj_row, jnp.zeros((pad,), jnp.int32)]).reshape(NNZ_PAD // 128, 128)
    colp = jnp.concatenate(
        [adj_col, jnp.zeros((pad,), jnp.int32)]).reshape(NNZ_PAD // 128, 128)
    valp = jnp.concatenate(
        [adj_val, jnp.zeros((pad,), jnp.float32)]).reshape(NNZ_PAD // 128, 128)

    spart, cid = _spmv_kernel(ego, slots, rowp, colp, valp)
    outc = _assemble_kernel(ego, slots.reshape(64, 96),
                            cid.reshape(64, 96), spart)
    return outc[:1024], outc[1024:2048], outc[2048:]
